# baseline (speedup 1.0000x reference)
import functools

import jax
import jax.numpy as jnp
from jax import lax
from jax.experimental import pallas as pl
from jax.experimental.pallas import tpu as pltpu
from jax.experimental.pallas import tpu_sc as plsc

_B, _C, _NPC, _D = 64, 256, 64, 64
_NC, _NS = 2, 16
_NW = _NC * _NS
_BPW = _B // _NW
_L = 16
_CT, _CS = _C // 128, 128
_DT, _DS = _D // 8, 8
_NCG = _C // _L
_JS = 24
_JB = 8
_RB = 8


def _sqrt16(x):
  x = jnp.maximum(x, 1e-12)
  i = plsc.bitcast(x, jnp.int32)
  y = plsc.bitcast(jnp.int32(0x5F3759DF) - (i >> 1), jnp.float32)
  y = y * (1.5 - 0.5 * x * y * y)
  y = y * (1.5 - 0.5 * x * y * y)
  return x * y


def _build_sc():
  mesh = plsc.VectorSubcoreMesh(
      core_axis_name="core", subcore_axis_name="sub",
      num_cores=_NC, num_subcores=_NS)

  @functools.partial(
      pl.kernel,
      out_type=jax.ShapeDtypeStruct((_B, _C), jnp.float32),
      mesh=mesh,
      compiler_params=pltpu.CompilerParams(needs_layout_passes=False),
      scratch_types=[
          pltpu.VMEM((2, _DT, _CT, _DS, _CS), jnp.float32),
          pltpu.VMEM((_D,), jnp.float32),
          pltpu.VMEM((_D, _L), jnp.float32),
          pltpu.VMEM((_L,), jnp.float32),
          pltpu.VMEM((_NCG, _L), jnp.float32),
          pltpu.VMEM((_C,), jnp.float32),
          pltpu.SemaphoreType.DMA,
          pltpu.SemaphoreType.DMA,
      ],
  )
  def ker(probes_hbm, emb6_hbm, kw_hbm, out_hbm,
          ebuf, pbuf, pbc, kwbuf, accbuf, orow, sem0, sem1):
    wid = lax.axis_index("core") * _NS + lax.axis_index("sub")
    pltpu.sync_copy(kw_hbm, kwbuf)
    neg_inv_kw = kwbuf[...]
    zero = jnp.zeros((_L,), jnp.float32)
    sems = (sem0, sem1)

    for bi in range(_BPW):
      b = wid * _BPW + bi
      pltpu.sync_copy(probes_hbm.at[b], pbuf)

      @plsc.parallel_loop(0, _D, unroll=8)
      def _mk_splat(d):
        pbc[d] = plsc.load_gather(
            pbuf, [jnp.full((_L,), d, jnp.int32)]) + 1e-6

      for cg in range(_NCG):
        accbuf[cg] = zero

      pltpu.async_copy(emb6_hbm.at[b, 0], ebuf.at[0], sem0)
      pltpu.async_copy(emb6_hbm.at[b, 1], ebuf.at[1], sem1)

      def slab_pair(t, carry, b=b):
        for par in range(2):
          j = 2 * t + par
          pltpu.make_async_copy(
              emb6_hbm.at[b, j], ebuf.at[par], sems[par]).wait()

          for half in range(2):
            cgs = list(range(half * 8, half * 8 + 8))

            @plsc.parallel_loop(0, _DT, unroll=8, carry=(zero,) * 8)
            def _dt_body(dt, d2, par=par, cgs=cgs):
              d2 = list(d2)
              for ds in range(_DS):
                p = pbc[dt * _DS + ds]
                for i, cg in enumerate(cgs):
                  ct, csb = cg // 8, cg % 8
                  e = ebuf[par, dt, ct, ds, pl.ds(csb * _L, _L)]
                  dfr = p - e
                  d2[i] = d2[i] + dfr * dfr
              return tuple(d2)

            for i, cg in enumerate(cgs):
              accbuf[cg] = accbuf[cg] + jnp.exp(
                  _sqrt16(_dt_body[i]) * neg_inv_kw)

          @pl.when(j + 2 < _JS)
          def _(b=b, j=j, par=par):
            pltpu.async_copy(
                emb6_hbm.at[b, j + 2], ebuf.at[par], sems[par])
        return carry

      lax.fori_loop(0, _JS // 2, slab_pair, 0)
      for cg in range(_NCG):
        orow[pl.ds(cg * _L, _L)] = accbuf[cg] * (1.0 / _NPC)
      pltpu.sync_copy(orow, out_hbm.at[b])

  return ker


def _tc_body(p_ref, e_ref, kw_ref, o_ref):
  ji = pl.program_id(1)

  @pl.when(ji == 0)
  def _():
    o_ref[...] = jnp.zeros_like(o_ref)

  nik = kw_ref[0, 0]
  p2 = p_ref[...] + 1e-6
  for ct in range(_CT):
    s = jnp.zeros((_RB, _CS), jnp.float32)
    for j in range(_JB):
      d2 = jnp.zeros((_RB, _DS, _CS), jnp.float32)
      for dt in range(_DT):
        e = e_ref[:, j, dt, ct]
        dfr = p2[:, :, dt][..., None] - e
        d2 = d2 + dfr * dfr
      d2s = jnp.sum(d2, axis=1)
      s = s + jnp.exp(jnp.sqrt(d2s) * nik)
    o_ref[:, ct * _CS:(ct + 1) * _CS] += s * (1.0 / _NPC)


def _build_tc():
  joff = _JS // _JB
  return pl.pallas_call(
      _tc_body,
      grid=(_B // _RB, (_NPC - _JS) // _JB),
      in_specs=[
          pl.BlockSpec((_RB, _DS, _DT), lambda bi, ji: (bi, 0, 0)),
          pl.BlockSpec((_RB, _JB, _DT, _CT, _DS, _CS),
                       lambda bi, ji: (bi, ji + joff, 0, 0, 0, 0)),
          pl.BlockSpec(memory_space=pltpu.SMEM),
      ],
      out_specs=pl.BlockSpec((_RB, _C), lambda bi, ji: (bi, 0)),
      out_shape=jax.ShapeDtypeStruct((_B, _C), jnp.float32),
      compiler_params=pltpu.CompilerParams(
          dimension_semantics=("parallel", "arbitrary")),
  )


_KER_SC = _build_sc()
_KER_TC = _build_tc()


def kernel(probes, emb_mats, kernel_width):
  emb6 = jnp.transpose(
      emb_mats.reshape(_B, _CT, _CS, _NPC, _DT, _DS), (0, 3, 4, 1, 5, 2))
  neg_inv_kw1 = (-1.0 / kernel_width[0]).astype(jnp.float32)
  neg_inv_kw16 = jnp.broadcast_to(neg_inv_kw1, (_L,))
  probes_t = jnp.swapaxes(probes.reshape(_B, _DT, _DS), 1, 2)
  out_tc = _KER_TC(probes_t, emb6, neg_inv_kw1.reshape(1, 1))
  out_sc = _KER_SC(probes, emb6, neg_inv_kw16)
  return out_tc + out_sc


# speedup vs baseline: 1.3223x; 1.3223x over previous
import functools

import jax
import jax.numpy as jnp
from jax import lax
from jax.experimental import pallas as pl
from jax.experimental.pallas import tpu as pltpu
from jax.experimental.pallas import tpu_sc as plsc

_B, _C, _NPC, _D = 64, 256, 64, 64
_NC, _NS = 2, 16
_NW = _NC * _NS
_BPW = _B // _NW
_L = 16
_CT, _CS = _C // 128, 128
_DT, _DS = _D // 8, 8
_NCG = _C // _L
_JS = 24
_JB = 8
_RB = 8


def _sqrt16(x):
  x = jnp.maximum(x, 1e-12)
  i = plsc.bitcast(x, jnp.int32)
  y = plsc.bitcast(jnp.int32(0x5F3759DF) - (i >> 1), jnp.float32)
  y = y * (1.5 - 0.5 * x * y * y)
  y = y * (1.5 - 0.5 * x * y * y)
  return x * y


def _build_sc():
  mesh = plsc.VectorSubcoreMesh(
      core_axis_name="core", subcore_axis_name="sub",
      num_cores=_NC, num_subcores=_NS)

  @functools.partial(
      pl.kernel,
      out_type=jax.ShapeDtypeStruct((_B, _C), jnp.float32),
      mesh=mesh,
      compiler_params=pltpu.CompilerParams(needs_layout_passes=False),
      scratch_types=[
          pltpu.VMEM((2, _DT, _CT, _DS, _CS), jnp.float32),
          pltpu.VMEM((_D,), jnp.float32),
          pltpu.VMEM((_D, _L), jnp.float32),
          pltpu.VMEM((_L,), jnp.float32),
          pltpu.VMEM((_NCG, _L), jnp.float32),
          pltpu.VMEM((_C,), jnp.float32),
          pltpu.SemaphoreType.DMA,
          pltpu.SemaphoreType.DMA,
      ],
  )
  def ker(probes_hbm, emb6_hbm, kw_hbm, out_hbm,
          ebuf, pbuf, pbc, kwbuf, accbuf, orow, sem0, sem1):
    wid = lax.axis_index("core") * _NS + lax.axis_index("sub")
    pltpu.sync_copy(kw_hbm, kwbuf)
    neg_inv_kw = kwbuf[...]
    zero = jnp.zeros((_L,), jnp.float32)
    sems = (sem0, sem1)

    for bi in range(_BPW):
      b = wid * _BPW + bi
      pltpu.sync_copy(probes_hbm.at[b], pbuf)

      @plsc.parallel_loop(0, _D, unroll=8)
      def _mk_splat(d):
        pbc[d] = plsc.load_gather(
            pbuf, [jnp.full((_L,), d, jnp.int32)]) + 1e-6

      for cg in range(_NCG):
        accbuf[cg] = zero

      pltpu.async_copy(emb6_hbm.at[b, 0], ebuf.at[0], sem0)
      pltpu.async_copy(emb6_hbm.at[b, 1], ebuf.at[1], sem1)

      def slab_pair(t, carry, b=b):
        for par in range(2):
          j = 2 * t + par
          pltpu.make_async_copy(
              emb6_hbm.at[b, j], ebuf.at[par], sems[par]).wait()

          for half in range(2):
            cgs = list(range(half * 8, half * 8 + 8))

            @plsc.parallel_loop(0, _DT, unroll=4, carry=(zero,) * 8)
            def _dt_body(dt, d2, par=par, cgs=cgs):
              d2 = list(d2)
              for ds in range(_DS):
                p = pbc[dt * _DS + ds]
                for i, cg in enumerate(cgs):
                  ct, csb = cg // 8, cg % 8
                  e = ebuf[par, dt, ct, ds, pl.ds(csb * _L, _L)]
                  dfr = p - e
                  d2[i] = d2[i] + dfr * dfr
              return tuple(d2)

            for i, cg in enumerate(cgs):
              accbuf[cg] = accbuf[cg] + jnp.exp(
                  _sqrt16(_dt_body[i]) * neg_inv_kw)

          @pl.when(j + 2 < _JS)
          def _(b=b, j=j, par=par):
            pltpu.async_copy(
                emb6_hbm.at[b, j + 2], ebuf.at[par], sems[par])
        return carry

      lax.fori_loop(0, _JS // 2, slab_pair, 0)
      for cg in range(_NCG):
        orow[pl.ds(cg * _L, _L)] = accbuf[cg] * (1.0 / _NPC)
      pltpu.sync_copy(orow, out_hbm.at[b])

  return ker


def _tc_body(p_ref, e_ref, kw_ref, o_ref):
  ji = pl.program_id(1)

  @pl.when(ji == 0)
  def _():
    o_ref[...] = jnp.zeros_like(o_ref)

  nik = kw_ref[0, 0]
  p2 = p_ref[...] + 1e-6
  for ct in range(_CT):
    s = jnp.zeros((_RB, _CS), jnp.float32)
    for j in range(_JB):
      d2 = jnp.zeros((_RB, _DS, _CS), jnp.float32)
      for dt in range(_DT):
        e = e_ref[:, j, dt, ct]
        dfr = p2[:, :, dt][..., None] - e
        d2 = d2 + dfr * dfr
      d2s = jnp.sum(d2, axis=1)
      s = s + jnp.exp(jnp.sqrt(d2s) * nik)
    o_ref[:, ct * _CS:(ct + 1) * _CS] += s * (1.0 / _NPC)


def _build_tc():
  joff = _JS // _JB
  return pl.pallas_call(
      _tc_body,
      grid=(_B // _RB, (_NPC - _JS) // _JB),
      in_specs=[
          pl.BlockSpec((_RB, _DS, _DT), lambda bi, ji: (bi, 0, 0)),
          pl.BlockSpec((_RB, _JB, _DT, _CT, _DS, _CS),
                       lambda bi, ji: (bi, ji + joff, 0, 0, 0, 0)),
          pl.BlockSpec(memory_space=pltpu.SMEM),
      ],
      out_specs=pl.BlockSpec((_RB, _C), lambda bi, ji: (bi, 0)),
      out_shape=jax.ShapeDtypeStruct((_B, _C), jnp.float32),
      compiler_params=pltpu.CompilerParams(
          dimension_semantics=("parallel", "arbitrary")),
  )


_KER_SC = _build_sc()
_KER_TC = _build_tc()


def kernel(probes, emb_mats, kernel_width):
  emb6 = jnp.transpose(
      emb_mats.reshape(_B, _CT, _CS, _NPC, _DT, _DS), (0, 3, 4, 1, 5, 2))
  neg_inv_kw1 = (-1.0 / kernel_width[0]).astype(jnp.float32)
  neg_inv_kw16 = jnp.broadcast_to(neg_inv_kw1, (_L,))
  probes_t = jnp.swapaxes(probes.reshape(_B, _DT, _DS), 1, 2)
  out_tc = _KER_TC(probes_t, emb6, neg_inv_kw1.reshape(1, 1))
  out_sc = _KER_SC(probes, emb6, neg_inv_kw16)
  return out_tc + out_sc
